# Initial kernel scaffold; baseline (speedup 1.0000x reference)
import jax
import jax.numpy as jnp
from jax.experimental import pallas as pl


def kernel(input_ids, weight):
    raise NotImplementedError("write your pallas kernel here")


# speedup vs baseline: 1.8487x; 1.8487x over previous
import functools

import jax
import jax.numpy as jnp
from jax import lax
from jax.experimental import pallas as pl
from jax.experimental.pallas import tpu as pltpu
from jax.experimental.pallas import tpu_sc as plsc

_BATCH = 16384
_HIST = 50
_DIM = 64
_ROWLEN = 128
_TOT = _BATCH * _HIST
_NROW = _TOT // _ROWLEN
_K = 8

_info = plsc.get_sparse_core_info()
_NC, _NS = _info.num_cores, _info.num_subcores
_NW = _NC * _NS
_ROWS_PER_W = _NROW // _NW
_ITERS = _ROWS_PER_W // _K

_mesh = plsc.VectorSubcoreMesh(core_axis_name="c", subcore_axis_name="s")


@functools.partial(
    pl.kernel,
    mesh=_mesh,
    out_type=jax.ShapeDtypeStruct((_NROW, _ROWLEN, _DIM), jnp.float32),
    scratch_types=[
        pltpu.VMEM((_K, _ROWLEN), jnp.int32),
        pltpu.VMEM((_K, _ROWLEN, _DIM), jnp.float32),
        pltpu.SemaphoreType.DMA,
    ],
    compiler_params=pltpu.CompilerParams(use_tc_tiling_on_sc=False),
)
def _gather_kernel(table_hbm, idx_hbm, out_hbm, idx_v, rows_v, sem):
    wid = lax.axis_index("s") * _NC + lax.axis_index("c")
    base = wid * _ROWS_PER_W

    def body(i, carry):
        r0 = base + i * _K
        pltpu.sync_copy(idx_hbm.at[pl.ds(r0, _K)], idx_v)
        copies = [
            pltpu.async_copy(table_hbm.at[idx_v.at[j]], rows_v.at[j], sem)
            for j in range(_K)
        ]
        for c in copies:
            c.wait()
        pltpu.sync_copy(rows_v, out_hbm.at[pl.ds(r0, _K)])
        return carry

    lax.fori_loop(0, _ITERS, body, 0)


def kernel(input_ids, weight):
    idx = input_ids.reshape(_NROW, _ROWLEN).astype(jnp.int32)
    out = _gather_kernel(weight, idx)
    return out.reshape(_BATCH, _HIST, _DIM)


# speedup vs baseline: 1.8758x; 1.0147x over previous
import functools

import jax
import jax.numpy as jnp
from jax import lax
from jax.experimental import pallas as pl
from jax.experimental.pallas import tpu as pltpu
from jax.experimental.pallas import tpu_sc as plsc

_BATCH = 16384
_HIST = 50
_DIM = 64
_ROWLEN = 128
_TOT = _BATCH * _HIST
_NROW = _TOT // _ROWLEN
_K = 5

_info = plsc.get_sparse_core_info()
_NC, _NS = _info.num_cores, _info.num_subcores
_NW = _NC * _NS
_ROWS_PER_W = _NROW // _NW
_ITERS = _ROWS_PER_W // _K

_mesh = plsc.VectorSubcoreMesh(core_axis_name="c", subcore_axis_name="s")


@functools.partial(
    pl.kernel,
    mesh=_mesh,
    out_type=jax.ShapeDtypeStruct((_NROW, _ROWLEN, _DIM), jnp.float32),
    scratch_types=[
        pltpu.VMEM((2, _K, _ROWLEN), jnp.int32),
        pltpu.VMEM((2, _K, _ROWLEN, _DIM), jnp.float32),
        pltpu.SemaphoreType.DMA((2,)),
        pltpu.SemaphoreType.DMA((2,)),
    ],
    compiler_params=pltpu.CompilerParams(use_tc_tiling_on_sc=False),
)
def _gather_kernel(table_hbm, idx_hbm, out_hbm, idx_v, rows_v, gsem, ssem):
    wid = lax.axis_index("s") * _NC + lax.axis_index("c")
    base = wid * _ROWS_PER_W

    def fire(chunk, b):
        r0 = base + chunk * _K
        pltpu.sync_copy(idx_hbm.at[pl.ds(r0, _K)], idx_v.at[b])
        for j in range(_K):
            pltpu.async_copy(
                table_hbm.at[idx_v.at[b, j]], rows_v.at[b, j], gsem.at[b]
            )

    def drain_gathers(b):
        for j in range(_K):
            pltpu.make_async_copy(
                table_hbm.at[idx_v.at[b, j]], rows_v.at[b, j], gsem.at[b]
            ).wait()

    def store(chunk, b):
        r0 = base + chunk * _K
        return pltpu.async_copy(rows_v.at[b], out_hbm.at[pl.ds(r0, _K)], ssem.at[b])

    def drain_store(chunk, b):
        r0 = base + chunk * _K
        pltpu.make_async_copy(
            rows_v.at[b], out_hbm.at[pl.ds(r0, _K)], ssem.at[b]
        ).wait()

    fire(0, 0)
    def body(i2, carry):
        for p in (0, 1):
            i = 2 * i2 + p
            n = 1 - p
            @pl.when(i + 1 < _ITERS)
            def _prefetch():
                @pl.when(i >= 1)
                def _reuse():
                    drain_store(i - 1, n)
                fire(i + 1, n)
            drain_gathers(p)
            store(i, p)
        return carry

    lax.fori_loop(0, _ITERS // 2, body, 0)
    drain_store(_ITERS - 1, (_ITERS - 1) % 2)


def kernel(input_ids, weight):
    idx = input_ids.reshape(_NROW, _ROWLEN).astype(jnp.int32)
    out = _gather_kernel(weight, idx)
    return out.reshape(_BATCH, _HIST, _DIM)
